# Initial kernel scaffold; baseline (speedup 1.0000x reference)
import jax
import jax.numpy as jnp
from jax.experimental import pallas as pl


def kernel(features, gt_boxes, im_info, conv_w, conv_b, cls_w, cls_b, box_w, box_b):
    raise NotImplementedError("write your pallas kernel here")


# speedup vs baseline: 6.7204x; 6.7204x over previous
import jax
import jax.numpy as jnp
import numpy as np
from jax import lax
from jax.experimental import pallas as pl
from jax.experimental.pallas import tpu as pltpu

_N_ANCHORS = 9
_FEAT_STRIDE = 16
_PRE_NMS = 6000
_POST_NMS = 300
_NMS_THRESH = 0.7
_MIN_SIZE = 16
_PAD = 6016
_ROWS = _PAD // 128


def _whctrs(a):
    w = a[2] - a[0] + 1.0
    h = a[3] - a[1] + 1.0
    return w, h, a[0] + 0.5 * (w - 1), a[1] + 0.5 * (h - 1)


def _mkanchors(ws, hs, x, y):
    ws = ws[:, None]
    hs = hs[:, None]
    return np.hstack((x - 0.5 * (ws - 1), y - 0.5 * (hs - 1),
                      x + 0.5 * (ws - 1), y + 0.5 * (hs - 1)))


def _gen_anchors(base_size=16, ratios=(0.5, 1.0, 2.0), scales=(8, 16, 32)):
    base = np.array([1, 1, base_size, base_size], dtype=np.float64) - 1
    w, h, x, y = _whctrs(base)
    size = w * h
    ws = np.round(np.sqrt(size / np.array(ratios)))
    hs = np.round(ws * np.array(ratios))
    ra = _mkanchors(ws, hs, x, y)
    out = []
    for i in range(ra.shape[0]):
        w, h, x, y = _whctrs(ra[i])
        out.append(_mkanchors(w * np.array(scales), h * np.array(scales), x, y))
    return np.vstack(out).astype(np.float32)


def _conv(x, w, b, pad):
    y = lax.conv_general_dilated(x, w, (1, 1), [(pad, pad), (pad, pad)],
                                 dimension_numbers=('NCHW', 'OIHW', 'NCHW'))
    return y + b[None, :, None, None]


def _nms_body(x1_ref, y1_ref, x2_ref, y2_ref, s_ref, out_ref, s_scr):
    x1 = x1_ref[...]
    y1 = y1_ref[...]
    x2 = x2_ref[...]
    y2 = y2_ref[...]
    areas = (x2 - x1 + 1.0) * (y2 - y1 + 1.0)
    s_scr[...] = s_ref[...]
    ridx = lax.broadcasted_iota(jnp.int32, (_ROWS, 128), 0)
    cidx = lax.broadcasted_iota(jnp.int32, (_ROWS, 128), 1)
    fidx = ridx * 128 + cidx
    neg = jnp.float32(-jnp.inf)

    def body(i, carry):
        s = s_scr[...]
        m = jnp.max(s)
        j = jnp.min(jnp.where(s == m, fidx, jnp.int32(2**30)))
        pick = fidx == j
        x1j = jnp.max(jnp.where(pick, x1, neg))
        y1j = jnp.max(jnp.where(pick, y1, neg))
        x2j = jnp.max(jnp.where(pick, x2, neg))
        y2j = jnp.max(jnp.where(pick, y2, neg))
        aj = jnp.max(jnp.where(pick, areas, neg))
        xx1 = jnp.maximum(x1j, x1)
        yy1 = jnp.maximum(y1j, y1)
        xx2 = jnp.minimum(x2j, x2)
        yy2 = jnp.minimum(y2j, y2)
        iw = jnp.maximum(0.0, xx2 - xx1 + 1.0)
        ih = jnp.maximum(0.0, yy2 - yy1 + 1.0)
        inter = iw * ih
        ovr = inter / (aj + areas - inter)
        s_scr[...] = jnp.where(ovr > _NMS_THRESH, neg, s)
        lane = lax.broadcasted_iota(jnp.int32, (1, 128), 1)
        row = jnp.where(lane == 1, x1j,
              jnp.where(lane == 2, y1j,
              jnp.where(lane == 3, x2j,
              jnp.where(lane == 4, y2j, 0.0))))
        out_ref[pl.ds(i, 1), :] = row
        return carry

    lax.fori_loop(0, _POST_NMS, body, 0)


def _nms_pallas(props, scores):
    npad = _PAD - _PRE_NMS
    sp = jnp.concatenate([scores, jnp.full((npad,), -jnp.inf, jnp.float32)])
    x1 = jnp.concatenate([props[:, 0], jnp.zeros((npad,), jnp.float32)])
    y1 = jnp.concatenate([props[:, 1], jnp.zeros((npad,), jnp.float32)])
    x2 = jnp.concatenate([props[:, 2], jnp.full((npad,), -1.0, jnp.float32)])
    y2 = jnp.concatenate([props[:, 3], jnp.full((npad,), -1.0, jnp.float32)])
    args = [a.reshape(_ROWS, 128) for a in (x1, y1, x2, y2, sp)]
    out = pl.pallas_call(
        _nms_body,
        out_shape=jax.ShapeDtypeStruct((_POST_NMS, 128), jnp.float32),
        scratch_shapes=[pltpu.VMEM((_ROWS, 128), jnp.float32)],
    )(*args)
    return out[:, :5]


def kernel(features, gt_boxes, im_info, conv_w, conv_b, cls_w, cls_b, box_w, box_b):
    bsize, _, h, w = features.shape
    rpn_feat = jax.nn.relu(_conv(features, conv_w, conv_b, 1))
    cls_score = _conv(rpn_feat, cls_w, cls_b, 0)
    cls_prob = jax.nn.softmax(cls_score.reshape(bsize, 2, _N_ANCHORS, h, w),
                              axis=1).reshape(bsize, 2 * _N_ANCHORS, h, w)
    box_reg = _conv(rpn_feat, box_w, box_b, 0)

    scores = cls_prob[:, _N_ANCHORS:, :, :].transpose(0, 2, 3, 1).reshape(-1)
    deltas = box_reg.transpose(0, 2, 3, 1).reshape(-1, 4)

    anchors = jnp.asarray(_gen_anchors(_FEAT_STRIDE))
    sx = np.arange(w) * _FEAT_STRIDE
    sy = np.arange(h) * _FEAT_STRIDE
    sx, sy = np.meshgrid(sx, sy)
    shifts = jnp.asarray(np.stack([sx.ravel(), sy.ravel(), sx.ravel(), sy.ravel()],
                                  axis=1).astype(np.float32))
    all_anchors = (anchors[None, :, :] + shifts[:, None, :]).reshape(-1, 4)

    ws = all_anchors[:, 2] - all_anchors[:, 0] + 1.0
    hs = all_anchors[:, 3] - all_anchors[:, 1] + 1.0
    cx = all_anchors[:, 0] + 0.5 * ws
    cy = all_anchors[:, 1] + 0.5 * hs
    dx, dy, dw, dh = deltas[:, 0], deltas[:, 1], deltas[:, 2], deltas[:, 3]
    pcx = dx * ws + cx
    pcy = dy * hs + cy
    pw = jnp.exp(dw) * ws
    ph = jnp.exp(dh) * hs
    proposals = jnp.stack([pcx - 0.5 * pw, pcy - 0.5 * ph,
                           pcx + 0.5 * pw, pcy + 0.5 * ph], axis=1)
    H = im_info[0, 0]
    W = im_info[0, 1]
    proposals = jnp.stack([
        jnp.clip(proposals[:, 0], 0.0, W - 1.0),
        jnp.clip(proposals[:, 1], 0.0, H - 1.0),
        jnp.clip(proposals[:, 2], 0.0, W - 1.0),
        jnp.clip(proposals[:, 3], 0.0, H - 1.0)], axis=1)

    min_size = _MIN_SIZE * im_info[0, 2]
    pws = proposals[:, 2] - proposals[:, 0] + 1.0
    phs = proposals[:, 3] - proposals[:, 1] + 1.0
    valid = (pws >= min_size) & (phs >= min_size)
    scores = jnp.where(valid, scores, -1e9)

    top_scores, order = lax.top_k(scores, _PRE_NMS)
    props = proposals[order]
    return _nms_pallas(props, top_scores)


# speedup vs baseline: 7.2587x; 1.0801x over previous
import jax
import jax.numpy as jnp
import numpy as np
from jax import lax
from jax.experimental import pallas as pl
from jax.experimental.pallas import tpu as pltpu

_N_ANCHORS = 9
_FEAT_STRIDE = 16
_PRE_NMS = 6000
_POST_NMS = 300
_NMS_THRESH = 0.7
_MIN_SIZE = 16
_PAD = 6016
_ROWS = _PAD // 128


def _whctrs(a):
    w = a[2] - a[0] + 1.0
    h = a[3] - a[1] + 1.0
    return w, h, a[0] + 0.5 * (w - 1), a[1] + 0.5 * (h - 1)


def _mkanchors(ws, hs, x, y):
    ws = ws[:, None]
    hs = hs[:, None]
    return np.hstack((x - 0.5 * (ws - 1), y - 0.5 * (hs - 1),
                      x + 0.5 * (ws - 1), y + 0.5 * (hs - 1)))


def _gen_anchors(base_size=16, ratios=(0.5, 1.0, 2.0), scales=(8, 16, 32)):
    base = np.array([1, 1, base_size, base_size], dtype=np.float64) - 1
    w, h, x, y = _whctrs(base)
    size = w * h
    ws = np.round(np.sqrt(size / np.array(ratios)))
    hs = np.round(ws * np.array(ratios))
    ra = _mkanchors(ws, hs, x, y)
    out = []
    for i in range(ra.shape[0]):
        w, h, x, y = _whctrs(ra[i])
        out.append(_mkanchors(w * np.array(scales), h * np.array(scales), x, y))
    return np.vstack(out).astype(np.float32)


def _conv(x, w, b, pad):
    y = lax.conv_general_dilated(x, w, (1, 1), [(pad, pad), (pad, pad)],
                                 dimension_numbers=('NCHW', 'OIHW', 'NCHW'))
    return y + b[None, :, None, None]


def _nms_body(x1_ref, y1_ref, x2_ref, y2_ref, out_ref, aidx_scr):
    x1 = x1_ref[...]
    y1 = y1_ref[...]
    x2 = x2_ref[...]
    y2 = y2_ref[...]
    areas = (x2 - x1 + 1.0) * (y2 - y1 + 1.0)
    ridx = lax.broadcasted_iota(jnp.int32, (_ROWS, 128), 0)
    cidx = lax.broadcasted_iota(jnp.int32, (_ROWS, 128), 1)
    fidx = ridx * 128 + cidx
    big = jnp.int32(2**30)
    aidx_scr[...] = jnp.where(fidx < _PRE_NMS, fidx, big)
    lane1 = lax.broadcasted_iota(jnp.int32, (1, 128), 1)
    neg = jnp.float32(-jnp.inf)

    def body(i, carry):
        aidx = aidx_scr[...]
        jraw = jnp.min(aidx)
        j = jnp.where(jraw >= big, 0, jraw)
        r = j // 128
        c = j - r * 128
        pickl = lane1 == c
        x1j = jnp.max(jnp.where(pickl, x1_ref[pl.ds(r, 1), :], neg))
        y1j = jnp.max(jnp.where(pickl, y1_ref[pl.ds(r, 1), :], neg))
        x2j = jnp.max(jnp.where(pickl, x2_ref[pl.ds(r, 1), :], neg))
        y2j = jnp.max(jnp.where(pickl, y2_ref[pl.ds(r, 1), :], neg))
        aj = (x2j - x1j + 1.0) * (y2j - y1j + 1.0)
        xx1 = jnp.maximum(x1j, x1)
        yy1 = jnp.maximum(y1j, y1)
        xx2 = jnp.minimum(x2j, x2)
        yy2 = jnp.minimum(y2j, y2)
        iw = jnp.maximum(0.0, xx2 - xx1 + 1.0)
        ih = jnp.maximum(0.0, yy2 - yy1 + 1.0)
        inter = iw * ih
        ovr = inter / (aj + areas - inter)
        aidx_scr[...] = jnp.where(ovr > _NMS_THRESH, big, aidx)
        row = jnp.where(lane1 == 1, x1j,
              jnp.where(lane1 == 2, y1j,
              jnp.where(lane1 == 3, x2j,
              jnp.where(lane1 == 4, y2j, 0.0))))
        out_ref[pl.ds(i, 1), :] = row
        return carry

    lax.fori_loop(0, _POST_NMS, body, 0)


def _nms_pallas(props):
    npad = _PAD - _PRE_NMS
    x1 = jnp.concatenate([props[:, 0], jnp.zeros((npad,), jnp.float32)])
    y1 = jnp.concatenate([props[:, 1], jnp.zeros((npad,), jnp.float32)])
    x2 = jnp.concatenate([props[:, 2], jnp.full((npad,), -1.0, jnp.float32)])
    y2 = jnp.concatenate([props[:, 3], jnp.full((npad,), -1.0, jnp.float32)])
    args = [a.reshape(_ROWS, 128) for a in (x1, y1, x2, y2)]
    out = pl.pallas_call(
        _nms_body,
        out_shape=jax.ShapeDtypeStruct((_POST_NMS, 128), jnp.float32),
        scratch_shapes=[pltpu.VMEM((_ROWS, 128), jnp.int32)],
    )(*args)
    return out[:, :5]


def kernel(features, gt_boxes, im_info, conv_w, conv_b, cls_w, cls_b, box_w, box_b):
    bsize, _, h, w = features.shape
    rpn_feat = jax.nn.relu(_conv(features, conv_w, conv_b, 1))
    cls_score = _conv(rpn_feat, cls_w, cls_b, 0)
    cls_prob = jax.nn.softmax(cls_score.reshape(bsize, 2, _N_ANCHORS, h, w),
                              axis=1).reshape(bsize, 2 * _N_ANCHORS, h, w)
    box_reg = _conv(rpn_feat, box_w, box_b, 0)

    scores = cls_prob[:, _N_ANCHORS:, :, :].transpose(0, 2, 3, 1).reshape(-1)
    deltas = box_reg.transpose(0, 2, 3, 1).reshape(-1, 4)

    anchors = jnp.asarray(_gen_anchors(_FEAT_STRIDE))
    sx = np.arange(w) * _FEAT_STRIDE
    sy = np.arange(h) * _FEAT_STRIDE
    sx, sy = np.meshgrid(sx, sy)
    shifts = jnp.asarray(np.stack([sx.ravel(), sy.ravel(), sx.ravel(), sy.ravel()],
                                  axis=1).astype(np.float32))
    all_anchors = (anchors[None, :, :] + shifts[:, None, :]).reshape(-1, 4)

    ws = all_anchors[:, 2] - all_anchors[:, 0] + 1.0
    hs = all_anchors[:, 3] - all_anchors[:, 1] + 1.0
    cx = all_anchors[:, 0] + 0.5 * ws
    cy = all_anchors[:, 1] + 0.5 * hs
    dx, dy, dw, dh = deltas[:, 0], deltas[:, 1], deltas[:, 2], deltas[:, 3]
    pcx = dx * ws + cx
    pcy = dy * hs + cy
    pw = jnp.exp(dw) * ws
    ph = jnp.exp(dh) * hs
    proposals = jnp.stack([pcx - 0.5 * pw, pcy - 0.5 * ph,
                           pcx + 0.5 * pw, pcy + 0.5 * ph], axis=1)
    H = im_info[0, 0]
    W = im_info[0, 1]
    proposals = jnp.stack([
        jnp.clip(proposals[:, 0], 0.0, W - 1.0),
        jnp.clip(proposals[:, 1], 0.0, H - 1.0),
        jnp.clip(proposals[:, 2], 0.0, W - 1.0),
        jnp.clip(proposals[:, 3], 0.0, H - 1.0)], axis=1)

    min_size = _MIN_SIZE * im_info[0, 2]
    pws = proposals[:, 2] - proposals[:, 0] + 1.0
    phs = proposals[:, 3] - proposals[:, 1] + 1.0
    valid = (pws >= min_size) & (phs >= min_size)
    scores = jnp.where(valid, scores, -1e9)

    top_scores, order = lax.top_k(scores, _PRE_NMS)
    props = proposals[order]
    return _nms_pallas(props)


# speedup vs baseline: 8.3086x; 1.1446x over previous
import jax
import jax.numpy as jnp
import numpy as np
from jax import lax
from jax.experimental import pallas as pl
from jax.experimental.pallas import tpu as pltpu

_N_ANCHORS = 9
_FEAT_STRIDE = 16
_PRE_NMS = 6000
_POST_NMS = 300
_NMS_THRESH = 0.7
_MIN_SIZE = 16
_PAD = 6016
_ROWS = _PAD // 128


def _whctrs(a):
    w = a[2] - a[0] + 1.0
    h = a[3] - a[1] + 1.0
    return w, h, a[0] + 0.5 * (w - 1), a[1] + 0.5 * (h - 1)


def _mkanchors(ws, hs, x, y):
    ws = ws[:, None]
    hs = hs[:, None]
    return np.hstack((x - 0.5 * (ws - 1), y - 0.5 * (hs - 1),
                      x + 0.5 * (ws - 1), y + 0.5 * (hs - 1)))


def _gen_anchors(base_size=16, ratios=(0.5, 1.0, 2.0), scales=(8, 16, 32)):
    base = np.array([1, 1, base_size, base_size], dtype=np.float64) - 1
    w, h, x, y = _whctrs(base)
    size = w * h
    ws = np.round(np.sqrt(size / np.array(ratios)))
    hs = np.round(ws * np.array(ratios))
    ra = _mkanchors(ws, hs, x, y)
    out = []
    for i in range(ra.shape[0]):
        w, h, x, y = _whctrs(ra[i])
        out.append(_mkanchors(w * np.array(scales), h * np.array(scales), x, y))
    return np.vstack(out).astype(np.float32)


_PIX = 64 * 64
_PBLK = 256
_K9 = 9 * 256


def _conv_body(xm_ref, x0_ref, xp_ref, w9_ref, cb_ref, wh_ref, hb_ref, out_ref):
    i = pl.program_id(0)
    base = i * _PBLK + 128
    refs = (xm_ref, x0_ref, xp_ref)
    parts = []
    for t in range(9):
        dy, dx = t // 3 - 1, t % 3 - 1
        parts.append(refs[dx + 1][pl.ds(base + dy * 64, _PBLK), :])
    xc = jnp.concatenate(parts, axis=1)
    y = jnp.dot(xc, w9_ref[...], preferred_element_type=jnp.float32)
    y = jnp.maximum(y + cb_ref[...], 0.0)
    heads = jnp.dot(y, wh_ref[...], preferred_element_type=jnp.float32)
    out_ref[...] = heads + hb_ref[...]


def _conv_heads_pallas(features, conv_w, conv_b, cls_w, cls_b, box_w, box_b):
    x = features.reshape(256, _PIX).T
    col = (np.arange(_PIX) % 64)
    zrow = jnp.zeros((1, 256), jnp.float32)
    xm = jnp.where(jnp.asarray(col == 0)[:, None],
                   0.0, jnp.concatenate([zrow, x[:-1]], axis=0))
    xp = jnp.where(jnp.asarray(col == 63)[:, None],
                   0.0, jnp.concatenate([x[1:], zrow], axis=0))
    pad = ((128, 128), (0, 0))
    xm = jnp.pad(xm, pad)
    x0 = jnp.pad(x, pad)
    xp = jnp.pad(xp, pad)
    w9 = jnp.transpose(conv_w, (2, 3, 1, 0)).reshape(_K9, 512)
    wh = jnp.concatenate([cls_w[:, :, 0, 0].T, box_w[:, :, 0, 0].T], axis=1)
    hb = jnp.concatenate([cls_b, box_b])[None, :]
    cb = conv_b[None, :]
    xspec = pl.BlockSpec((_PIX + 256, 256), lambda i: (0, 0))
    return pl.pallas_call(
        _conv_body,
        grid=(_PIX // _PBLK,),
        in_specs=[
            xspec, xspec, xspec,
            pl.BlockSpec((_K9, 512), lambda i: (0, 0)),
            pl.BlockSpec((1, 512), lambda i: (0, 0)),
            pl.BlockSpec((512, 54), lambda i: (0, 0)),
            pl.BlockSpec((1, 54), lambda i: (0, 0)),
        ],
        out_specs=pl.BlockSpec((_PBLK, 54), lambda i: (i, 0)),
        out_shape=jax.ShapeDtypeStruct((_PIX, 54), jnp.float32),
    )(xm, x0, xp, w9, cb, wh, hb)


def _nms_body(x1_ref, y1_ref, x2_ref, y2_ref, out_ref, aidx_scr):
    x1 = x1_ref[...]
    y1 = y1_ref[...]
    x2 = x2_ref[...]
    y2 = y2_ref[...]
    areas = (x2 - x1 + 1.0) * (y2 - y1 + 1.0)
    ridx = lax.broadcasted_iota(jnp.int32, (_ROWS, 128), 0)
    cidx = lax.broadcasted_iota(jnp.int32, (_ROWS, 128), 1)
    fidx = ridx * 128 + cidx
    big = jnp.int32(2**30)
    aidx_scr[...] = jnp.where(fidx < _PRE_NMS, fidx, big)
    lane1 = lax.broadcasted_iota(jnp.int32, (1, 128), 1)
    neg = jnp.float32(-jnp.inf)

    def body(i, carry):
        aidx = aidx_scr[...]
        jraw = jnp.min(aidx)
        j = jnp.where(jraw >= big, 0, jraw)
        r = j // 128
        c = j - r * 128
        pickl = lane1 == c
        x1j = jnp.max(jnp.where(pickl, x1_ref[pl.ds(r, 1), :], neg))
        y1j = jnp.max(jnp.where(pickl, y1_ref[pl.ds(r, 1), :], neg))
        x2j = jnp.max(jnp.where(pickl, x2_ref[pl.ds(r, 1), :], neg))
        y2j = jnp.max(jnp.where(pickl, y2_ref[pl.ds(r, 1), :], neg))
        aj = (x2j - x1j + 1.0) * (y2j - y1j + 1.0)
        xx1 = jnp.maximum(x1j, x1)
        yy1 = jnp.maximum(y1j, y1)
        xx2 = jnp.minimum(x2j, x2)
        yy2 = jnp.minimum(y2j, y2)
        iw = jnp.maximum(0.0, xx2 - xx1 + 1.0)
        ih = jnp.maximum(0.0, yy2 - yy1 + 1.0)
        inter = iw * ih
        ovr = inter / (aj + areas - inter)
        aidx_scr[...] = jnp.where(ovr > _NMS_THRESH, big, aidx)
        row = jnp.where(lane1 == 1, x1j,
              jnp.where(lane1 == 2, y1j,
              jnp.where(lane1 == 3, x2j,
              jnp.where(lane1 == 4, y2j, 0.0))))
        out_ref[pl.ds(i, 1), :] = row
        return carry

    lax.fori_loop(0, _POST_NMS, body, 0)


def _nms_pallas(props):
    npad = _PAD - _PRE_NMS
    x1 = jnp.concatenate([props[:, 0], jnp.zeros((npad,), jnp.float32)])
    y1 = jnp.concatenate([props[:, 1], jnp.zeros((npad,), jnp.float32)])
    x2 = jnp.concatenate([props[:, 2], jnp.full((npad,), -1.0, jnp.float32)])
    y2 = jnp.concatenate([props[:, 3], jnp.full((npad,), -1.0, jnp.float32)])
    args = [a.reshape(_ROWS, 128) for a in (x1, y1, x2, y2)]
    out = pl.pallas_call(
        _nms_body,
        out_shape=jax.ShapeDtypeStruct((_POST_NMS, 128), jnp.float32),
        scratch_shapes=[pltpu.VMEM((_ROWS, 128), jnp.int32)],
    )(*args)
    return out[:, :5]


def kernel(features, gt_boxes, im_info, conv_w, conv_b, cls_w, cls_b, box_w, box_b):
    bsize, _, h, w = features.shape
    heads = _conv_heads_pallas(features, conv_w, conv_b, cls_w, cls_b,
                               box_w, box_b)
    bg = heads[:, :_N_ANCHORS]
    fg = heads[:, _N_ANCHORS:2 * _N_ANCHORS]
    mx = jnp.maximum(bg, fg)
    ebg = jnp.exp(bg - mx)
    efg = jnp.exp(fg - mx)
    scores = (efg / (ebg + efg)).reshape(-1)
    deltas = heads[:, 2 * _N_ANCHORS:].reshape(-1, 4)

    anchors = jnp.asarray(_gen_anchors(_FEAT_STRIDE))
    sx = np.arange(w) * _FEAT_STRIDE
    sy = np.arange(h) * _FEAT_STRIDE
    sx, sy = np.meshgrid(sx, sy)
    shifts = jnp.asarray(np.stack([sx.ravel(), sy.ravel(), sx.ravel(), sy.ravel()],
                                  axis=1).astype(np.float32))
    all_anchors = (anchors[None, :, :] + shifts[:, None, :]).reshape(-1, 4)

    ws = all_anchors[:, 2] - all_anchors[:, 0] + 1.0
    hs = all_anchors[:, 3] - all_anchors[:, 1] + 1.0
    cx = all_anchors[:, 0] + 0.5 * ws
    cy = all_anchors[:, 1] + 0.5 * hs
    dx, dy, dw, dh = deltas[:, 0], deltas[:, 1], deltas[:, 2], deltas[:, 3]
    pcx = dx * ws + cx
    pcy = dy * hs + cy
    pw = jnp.exp(dw) * ws
    ph = jnp.exp(dh) * hs
    proposals = jnp.stack([pcx - 0.5 * pw, pcy - 0.5 * ph,
                           pcx + 0.5 * pw, pcy + 0.5 * ph], axis=1)
    H = im_info[0, 0]
    W = im_info[0, 1]
    proposals = jnp.stack([
        jnp.clip(proposals[:, 0], 0.0, W - 1.0),
        jnp.clip(proposals[:, 1], 0.0, H - 1.0),
        jnp.clip(proposals[:, 2], 0.0, W - 1.0),
        jnp.clip(proposals[:, 3], 0.0, H - 1.0)], axis=1)

    min_size = _MIN_SIZE * im_info[0, 2]
    pws = proposals[:, 2] - proposals[:, 0] + 1.0
    phs = proposals[:, 3] - proposals[:, 1] + 1.0
    valid = (pws >= min_size) & (phs >= min_size)
    scores = jnp.where(valid, scores, -1e9)

    top_scores, order = lax.top_k(scores, _PRE_NMS)
    props = proposals[order]
    return _nms_pallas(props)


# speedup vs baseline: 11.9257x; 1.4354x over previous
import jax
import jax.numpy as jnp
import numpy as np
from jax import lax
from jax.experimental import pallas as pl
from jax.experimental.pallas import tpu as pltpu

_N_ANCHORS = 9
_FEAT_STRIDE = 16
_PRE_NMS = 6000
_POST_NMS = 300
_NMS_THRESH = 0.7
_MIN_SIZE = 16
_PAD = 6016
_ROWS = _PAD // 128


def _whctrs(a):
    w = a[2] - a[0] + 1.0
    h = a[3] - a[1] + 1.0
    return w, h, a[0] + 0.5 * (w - 1), a[1] + 0.5 * (h - 1)


def _mkanchors(ws, hs, x, y):
    ws = ws[:, None]
    hs = hs[:, None]
    return np.hstack((x - 0.5 * (ws - 1), y - 0.5 * (hs - 1),
                      x + 0.5 * (ws - 1), y + 0.5 * (hs - 1)))


def _gen_anchors(base_size=16, ratios=(0.5, 1.0, 2.0), scales=(8, 16, 32)):
    base = np.array([1, 1, base_size, base_size], dtype=np.float64) - 1
    w, h, x, y = _whctrs(base)
    size = w * h
    ws = np.round(np.sqrt(size / np.array(ratios)))
    hs = np.round(ws * np.array(ratios))
    ra = _mkanchors(ws, hs, x, y)
    out = []
    for i in range(ra.shape[0]):
        w, h, x, y = _whctrs(ra[i])
        out.append(_mkanchors(w * np.array(scales), h * np.array(scales), x, y))
    return np.vstack(out).astype(np.float32)


_PIX = 64 * 64
_PBLK = 256
_K9 = 9 * 256


def _conv_body(info_ref, xm_ref, x0_ref, xp_ref, w9_ref, cb_ref, wh_ref,
               hb_ref, anc_ref, s_out, x1_out, y1_out, x2_out, y2_out):
    i = pl.program_id(0)
    base = i * _PBLK + 128
    refs = (xm_ref, x0_ref, xp_ref)
    parts = []
    for t in range(9):
        dy, dx = t // 3 - 1, t % 3 - 1
        parts.append(refs[dx + 1][pl.ds(base + dy * 64, _PBLK), :])
    xc = jnp.concatenate(parts, axis=1)
    y = jnp.dot(xc, w9_ref[...], preferred_element_type=jnp.float32)
    y = jnp.maximum(y + cb_ref[...], 0.0)
    heads = jnp.dot(y, wh_ref[...], preferred_element_type=jnp.float32)
    heads = heads + hb_ref[...]

    bg = heads[:, :_N_ANCHORS]
    fg = heads[:, _N_ANCHORS:2 * _N_ANCHORS]
    mx = jnp.maximum(bg, fg)
    ebg = jnp.exp(bg - mx)
    efg = jnp.exp(fg - mx)
    prob = efg / (ebg + efg)

    deltas = heads[:, 2 * _N_ANCHORS:]
    dx_ = deltas[:, 0:9]
    dy_ = deltas[:, 9:18]
    dw_ = deltas[:, 18:27]
    dh_ = deltas[:, 27:36]
    sub = lax.broadcasted_iota(jnp.int32, (_PBLK, 1), 0)
    p = i * _PBLK + sub
    sx = ((p % 64) * _FEAT_STRIDE).astype(jnp.float32)
    sy = ((p // 64) * _FEAT_STRIDE).astype(jnp.float32)
    ax1 = anc_ref[0:1, :] + sx
    ay1 = anc_ref[1:2, :] + sy
    ax2 = anc_ref[2:3, :] + sx
    ay2 = anc_ref[3:4, :] + sy
    ws = ax2 - ax1 + 1.0
    hs = ay2 - ay1 + 1.0
    cx = ax1 + 0.5 * ws
    cy = ay1 + 0.5 * hs
    pcx = dx_ * ws + cx
    pcy = dy_ * hs + cy
    pw = jnp.exp(dw_) * ws
    ph = jnp.exp(dh_) * hs
    H = info_ref[0]
    W = info_ref[1]
    x1 = jnp.clip(pcx - 0.5 * pw, 0.0, W - 1.0)
    y1 = jnp.clip(pcy - 0.5 * ph, 0.0, H - 1.0)
    x2 = jnp.clip(pcx + 0.5 * pw, 0.0, W - 1.0)
    y2 = jnp.clip(pcy + 0.5 * ph, 0.0, H - 1.0)
    pws = x2 - x1 + 1.0
    phs = y2 - y1 + 1.0
    min_size = _MIN_SIZE * info_ref[2]
    valid = (pws >= min_size) & (phs >= min_size)
    s_out[...] = jnp.where(valid, prob, -1e9)
    x1_out[...] = x1
    y1_out[...] = y1
    x2_out[...] = x2
    y2_out[...] = y2


def _conv_heads_pallas(features, im_info, conv_w, conv_b, cls_w, cls_b,
                       box_w, box_b):
    x = features.reshape(256, _PIX).T
    col = (np.arange(_PIX) % 64)
    zrow = jnp.zeros((1, 256), jnp.float32)
    xm = jnp.where(jnp.asarray(col == 0)[:, None],
                   0.0, jnp.concatenate([zrow, x[:-1]], axis=0))
    xp = jnp.where(jnp.asarray(col == 63)[:, None],
                   0.0, jnp.concatenate([x[1:], zrow], axis=0))
    pad = ((128, 128), (0, 0))
    xm = jnp.pad(xm, pad)
    x0 = jnp.pad(x, pad)
    xp = jnp.pad(xp, pad)
    w9 = jnp.transpose(conv_w, (2, 3, 1, 0)).reshape(_K9, 512)
    perm = np.array([a * 4 + c for c in range(4) for a in range(_N_ANCHORS)])
    wh = jnp.concatenate([cls_w[:, :, 0, 0].T, box_w[perm, :, 0, 0].T], axis=1)
    hb = jnp.concatenate([cls_b, box_b[perm]])[None, :]
    cb = conv_b[None, :]
    anc = jnp.asarray(_gen_anchors(_FEAT_STRIDE).T)
    info = im_info[0]
    xspec = pl.BlockSpec((_PIX + 256, 256), lambda i: (0, 0))
    oshape = jax.ShapeDtypeStruct((_PIX, _N_ANCHORS), jnp.float32)
    ospec = pl.BlockSpec((_PBLK, _N_ANCHORS), lambda i: (i, 0))
    return pl.pallas_call(
        _conv_body,
        grid=(_PIX // _PBLK,),
        in_specs=[
            pl.BlockSpec(memory_space=pltpu.SMEM),
            xspec, xspec, xspec,
            pl.BlockSpec((_K9, 512), lambda i: (0, 0)),
            pl.BlockSpec((1, 512), lambda i: (0, 0)),
            pl.BlockSpec((512, 54), lambda i: (0, 0)),
            pl.BlockSpec((1, 54), lambda i: (0, 0)),
            pl.BlockSpec((4, _N_ANCHORS), lambda i: (0, 0)),
        ],
        out_specs=[ospec] * 5,
        out_shape=[oshape] * 5,
    )(info, xm, x0, xp, w9, cb, wh, hb, anc)


def _nms_body(x1_ref, y1_ref, x2_ref, y2_ref, out_ref, aidx_scr):
    x1 = x1_ref[...]
    y1 = y1_ref[...]
    x2 = x2_ref[...]
    y2 = y2_ref[...]
    areas = (x2 - x1 + 1.0) * (y2 - y1 + 1.0)
    ridx = lax.broadcasted_iota(jnp.int32, (_ROWS, 128), 0)
    cidx = lax.broadcasted_iota(jnp.int32, (_ROWS, 128), 1)
    fidx = ridx * 128 + cidx
    big = jnp.int32(2**30)
    aidx_scr[...] = jnp.where(fidx < _PRE_NMS, fidx, big)
    lane1 = lax.broadcasted_iota(jnp.int32, (1, 128), 1)
    neg = jnp.float32(-jnp.inf)

    def body(i, carry):
        aidx = aidx_scr[...]
        jraw = jnp.min(aidx)
        j = jnp.where(jraw >= big, 0, jraw)
        r = j // 128
        c = j - r * 128
        pickl = lane1 == c
        x1j = jnp.max(jnp.where(pickl, x1_ref[pl.ds(r, 1), :], neg))
        y1j = jnp.max(jnp.where(pickl, y1_ref[pl.ds(r, 1), :], neg))
        x2j = jnp.max(jnp.where(pickl, x2_ref[pl.ds(r, 1), :], neg))
        y2j = jnp.max(jnp.where(pickl, y2_ref[pl.ds(r, 1), :], neg))
        aj = (x2j - x1j + 1.0) * (y2j - y1j + 1.0)
        xx1 = jnp.maximum(x1j, x1)
        yy1 = jnp.maximum(y1j, y1)
        xx2 = jnp.minimum(x2j, x2)
        yy2 = jnp.minimum(y2j, y2)
        iw = jnp.maximum(0.0, xx2 - xx1 + 1.0)
        ih = jnp.maximum(0.0, yy2 - yy1 + 1.0)
        inter = iw * ih
        ovr = inter / (aj + areas - inter)
        aidx_scr[...] = jnp.where(ovr > _NMS_THRESH, big, aidx)
        row = jnp.where(lane1 == 1, x1j,
              jnp.where(lane1 == 2, y1j,
              jnp.where(lane1 == 3, x2j,
              jnp.where(lane1 == 4, y2j, 0.0))))
        out_ref[pl.ds(i, 1), :] = row
        return carry

    lax.fori_loop(0, _POST_NMS, body, 0)


def _nms_pallas(props):
    npad = _PAD - _PRE_NMS
    x1 = jnp.concatenate([props[:, 0], jnp.zeros((npad,), jnp.float32)])
    y1 = jnp.concatenate([props[:, 1], jnp.zeros((npad,), jnp.float32)])
    x2 = jnp.concatenate([props[:, 2], jnp.full((npad,), -1.0, jnp.float32)])
    y2 = jnp.concatenate([props[:, 3], jnp.full((npad,), -1.0, jnp.float32)])
    args = [a.reshape(_ROWS, 128) for a in (x1, y1, x2, y2)]
    out = pl.pallas_call(
        _nms_body,
        out_shape=jax.ShapeDtypeStruct((_POST_NMS, 128), jnp.float32),
        scratch_shapes=[pltpu.VMEM((_ROWS, 128), jnp.int32)],
    )(*args)
    return out[:, :5]


def kernel(features, gt_boxes, im_info, conv_w, conv_b, cls_w, cls_b, box_w, box_b):
    s, x1, y1, x2, y2 = _conv_heads_pallas(features, im_info, conv_w, conv_b,
                                           cls_w, cls_b, box_w, box_b)
    scores = s.reshape(-1)
    proposals = jnp.stack([x1.reshape(-1), y1.reshape(-1),
                           x2.reshape(-1), y2.reshape(-1)], axis=1)
    top_scores, order = lax.top_k(scores, _PRE_NMS)
    props = proposals[order]
    return _nms_pallas(props)
